# baseline (speedup 1.0000x reference)
import functools

import jax
import jax.numpy as jnp
from jax import lax
from jax.experimental import pallas as pl
from jax.experimental.pallas import tpu as pltpu
from jax.experimental.pallas import tpu_sc as plsc

LANES = 16


@functools.partial(jax.jit, static_argnums=())
def kernel(x, table, pe):
    B, S = x.shape
    V, D = table.shape
    pe = pe[:S]

    NC, NS = 2, 16
    SB = S // NS
    BPW = B // NC
    assert S % NS == 0 and SB % 128 == 0 and B % NC == 0 and D % LANES == 0

    mesh = plsc.VectorSubcoreMesh(core_axis_name="c", subcore_axis_name="s")

    NBUF = 10
    PREF = 7
    UN = 64
    UPB = SB // UN
    UNITS = BPW * UPB

    @functools.partial(
        pl.kernel,
        out_type=jax.ShapeDtypeStruct((B, S, D), jnp.float32),
        mesh=mesh,
        scratch_types=[
            pltpu.VMEM((BPW, SB), jnp.int32),
            pltpu.VMEM((SB, D), jnp.float32),
            pltpu.VMEM((NBUF, UN, D), jnp.float32),
            [pltpu.SemaphoreType.DMA] * NBUF,
            [pltpu.SemaphoreType.DMA] * NBUF,
            pltpu.SemaphoreType.DMA,
        ],
    )
    def emb_kernel(x_hbm, table_hbm, pe_hbm, out_hbm,
                   idx_v, pe_v, rows_v, gsem, ssem, psem):
        c = lax.axis_index("c")
        s = lax.axis_index("s")
        sbase = s * SB
        bbase = c * BPW
        pe_h = pltpu.async_copy(pe_hbm.at[pl.ds(sbase, SB), :], pe_v, psem)
        pltpu.sync_copy(x_hbm.at[pl.ds(bbase, BPW), pl.ds(sbase, SB)], idx_v)

        def gather(u):
            k = u % NBUF
            b, off = u // UPB, (u % UPB) * UN
            return pltpu.async_copy(
                table_hbm.at[idx_v.at[b, pl.ds(off, UN)]], rows_v.at[k],
                gsem[k])

        def store(u):
            k = u % NBUF
            b, off = u // UPB, (u % UPB) * UN
            return pltpu.async_copy(
                rows_v.at[k],
                out_hbm.at[bbase + b, pl.ds(sbase + off, UN), :], ssem[k])

        def add_pe(u):
            k = u % NBUF
            off = (u % UPB) * UN

            def row_body(r, cc):
                for j in range(D // LANES):
                    sl = pl.ds(j * LANES, LANES)
                    plsc.addupdate(rows_v.at[k, r, sl], pe_v[off + r, sl])
                return cc

            lax.fori_loop(0, UN, row_body, 0)

        gh = [None] * UNITS
        sh = [None] * UNITS
        waited = [False] * UNITS
        for u in range(min(PREF, UNITS)):
            gh[u] = gather(u)
        pe_h.wait()
        for u in range(UNITS):
            gh[u].wait()
            add_pe(u)
            sh[u] = store(u)
            nxt = u + PREF
            if nxt < UNITS:
                prev = nxt - NBUF
                if prev >= 0:
                    sh[prev].wait()
                    waited[prev] = True
                gh[nxt] = gather(nxt)
        for u in range(UNITS):
            if not waited[u]:
                sh[u].wait()

    return emb_kernel(x, table, pe)


# speedup vs baseline: 1.0205x; 1.0205x over previous
import functools

import jax
import jax.numpy as jnp
from jax import lax
from jax.experimental import pallas as pl
from jax.experimental.pallas import tpu as pltpu
from jax.experimental.pallas import tpu_sc as plsc

LANES = 16


@functools.partial(jax.jit, static_argnums=())
def kernel(x, table, pe):
    B, S = x.shape
    V, D = table.shape
    pe = pe[:S]

    NC, NS = 2, 16
    SB = S // NS
    BPW = B // NC
    assert S % NS == 0 and SB % 128 == 0 and B % NC == 0 and D % LANES == 0

    mesh = plsc.VectorSubcoreMesh(core_axis_name="c", subcore_axis_name="s")

    NBUF = 10
    PREF = 8
    UN = 64
    UPB = SB // UN
    UNITS = BPW * UPB

    @functools.partial(
        pl.kernel,
        out_type=jax.ShapeDtypeStruct((B, S, D), jnp.float32),
        mesh=mesh,
        scratch_types=[
            pltpu.VMEM((BPW, SB), jnp.int32),
            pltpu.VMEM((SB, D), jnp.float32),
            pltpu.VMEM((NBUF, UN, D), jnp.float32),
            [pltpu.SemaphoreType.DMA] * NBUF,
            [pltpu.SemaphoreType.DMA] * NBUF,
            pltpu.SemaphoreType.DMA,
        ],
    )
    def emb_kernel(x_hbm, table_hbm, pe_hbm, out_hbm,
                   idx_v, pe_v, rows_v, gsem, ssem, psem):
        c = lax.axis_index("c")
        s = lax.axis_index("s")
        sbase = s * SB
        bbase = c * BPW
        pe_h = pltpu.async_copy(pe_hbm.at[pl.ds(sbase, SB), :], pe_v, psem)
        pltpu.sync_copy(x_hbm.at[pl.ds(bbase, BPW), pl.ds(sbase, SB)], idx_v)

        def gather(u):
            k = u % NBUF
            b, off = u // UPB, (u % UPB) * UN
            return pltpu.async_copy(
                table_hbm.at[idx_v.at[b, pl.ds(off, UN)]], rows_v.at[k],
                gsem[k])

        def store(u):
            k = u % NBUF
            b, off = u // UPB, (u % UPB) * UN
            return pltpu.async_copy(
                rows_v.at[k],
                out_hbm.at[bbase + b, pl.ds(sbase + off, UN), :], ssem[k])

        def add_pe(u):
            k = u % NBUF
            off = (u % UPB) * UN

            def row_body(r, cc):
                for j in range(D // LANES):
                    sl = pl.ds(j * LANES, LANES)
                    plsc.addupdate(rows_v.at[k, r, sl], pe_v[off + r, sl])
                return cc

            lax.fori_loop(0, UN, row_body, 0)

        gh = [None] * UNITS
        sh = [None] * UNITS
        waited = [False] * UNITS
        for u in range(min(PREF, UNITS)):
            gh[u] = gather(u)
        pe_h.wait()
        for u in range(UNITS):
            gh[u].wait()
            add_pe(u)
            sh[u] = store(u)
            nxt = u + PREF
            if nxt < UNITS:
                prev = nxt - NBUF
                if prev >= 0:
                    sh[prev].wait()
                    waited[prev] = True
                gh[nxt] = gather(nxt)
        for u in range(UNITS):
            if not waited[u]:
                sh[u].wait()

    return emb_kernel(x, table, pe)
